# Initial kernel scaffold; baseline (speedup 1.0000x reference)
import jax
import jax.numpy as jnp
from jax.experimental import pallas as pl


def kernel(pred, target):
    raise NotImplementedError("write your pallas kernel here")


# speedup vs baseline: 9.0520x; 9.0520x over previous
import jax
import jax.numpy as jnp
import numpy as np
from jax import lax
from jax.experimental import pallas as pl
from jax.experimental.pallas import tpu as pltpu

Z, Y, X = 64, 64, 64
YX = Y * X
K = 1 << 18
INF = np.int32(2**31 - 1)
SIGN = np.int32(-(2**31))
BG_B = np.int32(3_200_000_000 - 2**31)


def _cc_and_zpass_kernel(target_ref, d1_ref, num_ref, lab_scr):
    tgt = target_ref[0]
    fg = tgt > 0.5
    colid = lax.broadcasted_iota(jnp.int32, (Z, YX), 1)
    rowid = lax.broadcasted_iota(jnp.int32, (Z, YX), 0)
    xpos = colid & (X - 1)
    ypos = colid >> 6
    flat = rowid * YX + colid

    lab_scr[...] = jnp.where(fg, flat, INF)

    inf_c1 = jnp.full((Z, 1), INF, jnp.int32)
    inf_c64 = jnp.full((Z, X), INF, jnp.int32)
    inf_r1 = jnp.full((1, YX), INF, jnp.int32)

    def prop(lab):
        a = jnp.concatenate([inf_c1, lab[:, :-1]], axis=1)
        a = jnp.where(xpos == 0, INF, a)
        b = jnp.concatenate([lab[:, 1:], inf_c1], axis=1)
        b = jnp.where(xpos == X - 1, INF, b)
        m = jnp.minimum(lab, jnp.minimum(a, b))
        a = jnp.concatenate([inf_c64, m[:, :-X]], axis=1)
        a = jnp.where(ypos == 0, INF, a)
        b = jnp.concatenate([m[:, X:], inf_c64], axis=1)
        b = jnp.where(ypos == Y - 1, INF, b)
        m = jnp.minimum(m, jnp.minimum(a, b))
        a = jnp.concatenate([inf_r1, m[:-1, :]], axis=0)
        b = jnp.concatenate([m[1:, :], inf_r1], axis=0)
        m = jnp.minimum(m, jnp.minimum(a, b))
        return jnp.where(fg, m, INF)

    def body(carry):
        lab = lab_scr[...]
        new = prop(lab)
        lab_scr[...] = new
        return jnp.max(jnp.where(new != lab, 1, 0))

    lax.while_loop(lambda c: c > 0, lambda c: body(c), jnp.int32(1))

    lab = lab_scr[...]
    rep = fg & (lab == flat)
    num_ref[...] = jnp.sum(rep.astype(jnp.int32)).reshape(1, 1, 1)

    lab_scr[...] = jnp.where(fg, lab ^ SIGN, BG_B)

    rvec = lax.broadcasted_iota(jnp.int32, (Z, 1), 0)
    d1_ref[0] = lab_scr[...]

    def zbody(j, _):
        cost = (rvec - j) * (rvec - j) * K
        cand = lab_scr[pl.ds(j, 1), :] + cost
        d1_ref[0] = jnp.minimum(d1_ref[0], cand)
        return 0

    lax.fori_loop(0, Z, zbody, 0)


def _minplus_kernel(din_ref, dout_ref):
    rvec = lax.broadcasted_iota(jnp.int32, (64, 1), 0)
    dout_ref[0] = din_ref[0]

    def body(j, _):
        cost = (rvec - j) * (rvec - j) * K
        cand = din_ref[0, pl.ds(j, 1), :] + cost
        dout_ref[0] = jnp.minimum(dout_ref[0], cand)
        return 0

    lax.fori_loop(0, 64, body, 0)


def _xpass_dice_kernel(din_ref, pred_ref, target_ref, num_ref, loss_ref, d_scr):
    rvec = lax.broadcasted_iota(jnp.int32, (64, 1), 0)
    d_scr[...] = din_ref[0]

    def body(j, _):
        cost = (rvec - j) * (rvec - j) * K
        cand = din_ref[0, pl.ds(j, 1), :] + cost
        d_scr[...] = jnp.minimum(d_scr[...], cand)
        return 0

    lax.fori_loop(0, 64, body, 0)

    root = d_scr[...] & (K - 1)
    ps = jax.nn.sigmoid(pred_ref[0])
    t = target_ref[0]
    pt = ps * t
    num = num_ref[0, 0, 0]

    def dice_body(_, carry):
        proc, acc = carry
        r = jnp.min(jnp.where(root > proc, root, INF))
        maskf = (root == r).astype(jnp.float32)
        inter = jnp.sum(pt * maskf)
        sp = jnp.sum(ps * maskf)
        sg = jnp.sum(t * maskf)
        dice = 2.0 * inter / (sp + sg + 1e-8)
        return r, acc + dice

    _, ssum = lax.fori_loop(0, num, dice_body, (jnp.int32(-1), jnp.float32(0.0)))
    loss_ref[...] = jnp.where(
        num == 0, jnp.float32(1.0), 1.0 - ssum / num.astype(jnp.float32)
    ).reshape(1, 1, 1)


def kernel(pred, target):
    B = pred.shape[0]
    tgt = target.reshape(B, Z, YX)

    d1, num = pl.pallas_call(
        _cc_and_zpass_kernel,
        grid=(B,),
        in_specs=[pl.BlockSpec((1, Z, YX), lambda b: (b, 0, 0))],
        out_specs=[
            pl.BlockSpec((1, Z, YX), lambda b: (b, 0, 0)),
            pl.BlockSpec((1, 1, 1), lambda b: (b, 0, 0)),
        ],
        out_shape=[
            jax.ShapeDtypeStruct((B, Z, YX), jnp.int32),
            jax.ShapeDtypeStruct((B, 1, 1), jnp.int32),
        ],
        scratch_shapes=[pltpu.VMEM((Z, YX), jnp.int32)],
    )(tgt)

    d1t = d1.reshape(B, Z, Y, X).transpose(0, 2, 3, 1).reshape(B, Y, X * Z)

    d2 = pl.pallas_call(
        _minplus_kernel,
        grid=(B,),
        in_specs=[pl.BlockSpec((1, Y, X * Z), lambda b: (b, 0, 0))],
        out_specs=pl.BlockSpec((1, Y, X * Z), lambda b: (b, 0, 0)),
        out_shape=jax.ShapeDtypeStruct((B, Y, X * Z), jnp.int32),
    )(d1t)

    d2t = d2.reshape(B, Y, X, Z).transpose(0, 2, 3, 1).reshape(B, X, Z * Y)
    pred_t = (
        pred.reshape(B, Z, Y, X).transpose(0, 3, 1, 2).reshape(B, X, Z * Y)
    )
    tgt_t = (
        target.reshape(B, Z, Y, X).transpose(0, 3, 1, 2).reshape(B, X, Z * Y)
    )

    losses = pl.pallas_call(
        _xpass_dice_kernel,
        grid=(B,),
        in_specs=[
            pl.BlockSpec((1, X, Z * Y), lambda b: (b, 0, 0)),
            pl.BlockSpec((1, X, Z * Y), lambda b: (b, 0, 0)),
            pl.BlockSpec((1, X, Z * Y), lambda b: (b, 0, 0)),
            pl.BlockSpec((1, 1, 1), lambda b: (b, 0, 0)),
        ],
        out_specs=pl.BlockSpec((1, 1, 1), lambda b: (b, 0, 0)),
        out_shape=jax.ShapeDtypeStruct((B, 1, 1), jnp.float32),
        scratch_shapes=[pltpu.VMEM((X, Z * Y), jnp.int32)],
    )(d2t, pred_t, tgt_t, num)

    return jnp.mean(losses)


# speedup vs baseline: 9.2159x; 1.0181x over previous
import jax
import jax.numpy as jnp
import numpy as np
from jax import lax
from jax.experimental import pallas as pl
from jax.experimental.pallas import tpu as pltpu

Z, Y, X = 64, 64, 64
YX = Y * X
K = 1 << 18
INF = np.int32(2**31 - 1)
SIGN = np.int32(-(2**31))
BG_B = np.int32(3_200_000_000 - 2**31)


def _cc_and_zpass_kernel(target_ref, d1_ref, num_ref, lab_scr):
    tgt = target_ref[0]
    fg = tgt > 0.5
    colid = lax.broadcasted_iota(jnp.int32, (Z, YX), 1)
    rowid = lax.broadcasted_iota(jnp.int32, (Z, YX), 0)
    xpos = colid & (X - 1)
    ypos = colid >> 6
    flat = rowid * YX + colid

    lab_scr[...] = jnp.where(fg, flat, INF)

    inf_c1 = jnp.full((Z, 1), INF, jnp.int32)
    inf_c64 = jnp.full((Z, X), INF, jnp.int32)
    inf_r1 = jnp.full((1, YX), INF, jnp.int32)

    def prop(lab):
        a = jnp.concatenate([inf_c1, lab[:, :-1]], axis=1)
        a = jnp.where(xpos == 0, INF, a)
        b = jnp.concatenate([lab[:, 1:], inf_c1], axis=1)
        b = jnp.where(xpos == X - 1, INF, b)
        m = jnp.minimum(lab, jnp.minimum(a, b))
        a = jnp.concatenate([inf_c64, m[:, :-X]], axis=1)
        a = jnp.where(ypos == 0, INF, a)
        b = jnp.concatenate([m[:, X:], inf_c64], axis=1)
        b = jnp.where(ypos == Y - 1, INF, b)
        m = jnp.minimum(m, jnp.minimum(a, b))
        a = jnp.concatenate([inf_r1, m[:-1, :]], axis=0)
        b = jnp.concatenate([m[1:, :], inf_r1], axis=0)
        m = jnp.minimum(m, jnp.minimum(a, b))
        return jnp.where(fg, m, INF)

    def body(carry):
        lab = lab_scr[...]
        new = prop(lab)
        lab_scr[...] = new
        return jnp.max(jnp.where(new != lab, 1, 0))

    lax.while_loop(lambda c: c > 0, lambda c: body(c), jnp.int32(1))

    lab = lab_scr[...]
    rep = fg & (lab == flat)
    num_ref[...] = jnp.sum(rep.astype(jnp.int32)).reshape(1, 1, 1)

    lab_scr[...] = jnp.where(fg, lab ^ SIGN, BG_B)

    _minplus_rows(lab_scr, None, d1_ref, 0)


TILE = 512


def _minplus_rows(src_ref, src_lead, dst_ref, dst_lead):
    rvec = lax.broadcasted_iota(jnp.int32, (64, 1), 0)
    for t in range(0, YX, TILE):
        sl = pl.ds(t, TILE)
        m = None
        for j in range(64):
            if src_lead is None:
                row = src_ref[pl.ds(j, 1), sl]
            else:
                row = src_ref[src_lead, pl.ds(j, 1), sl]
            cost = (rvec - j) * (rvec - j) * K
            cand = row + cost
            m = cand if m is None else jnp.minimum(m, cand)
        if dst_lead is None:
            dst_ref[:, sl] = m
        else:
            dst_ref[dst_lead, :, sl] = m


def _minplus_kernel(din_ref, dout_ref):
    _minplus_rows(din_ref, 0, dout_ref, 0)


def _xpass_dice_kernel(din_ref, pred_ref, target_ref, num_ref, loss_ref, d_scr):
    _minplus_rows(din_ref, 0, d_scr, None)

    root = d_scr[...] & (K - 1)
    ps = jax.nn.sigmoid(pred_ref[0])
    t = target_ref[0]
    pt = ps * t
    num = num_ref[0, 0, 0]

    def dice_body(_, carry):
        proc, acc = carry
        r = jnp.min(jnp.where(root > proc, root, INF))
        maskf = (root == r).astype(jnp.float32)
        inter = jnp.sum(pt * maskf)
        sp = jnp.sum(ps * maskf)
        sg = jnp.sum(t * maskf)
        dice = 2.0 * inter / (sp + sg + 1e-8)
        return r, acc + dice

    _, ssum = lax.fori_loop(0, num, dice_body, (jnp.int32(-1), jnp.float32(0.0)))
    loss_ref[...] = jnp.where(
        num == 0, jnp.float32(1.0), 1.0 - ssum / num.astype(jnp.float32)
    ).reshape(1, 1, 1)


def kernel(pred, target):
    B = pred.shape[0]
    tgt = target.reshape(B, Z, YX)

    d1, num = pl.pallas_call(
        _cc_and_zpass_kernel,
        grid=(B,),
        in_specs=[pl.BlockSpec((1, Z, YX), lambda b: (b, 0, 0))],
        out_specs=[
            pl.BlockSpec((1, Z, YX), lambda b: (b, 0, 0)),
            pl.BlockSpec((1, 1, 1), lambda b: (b, 0, 0)),
        ],
        out_shape=[
            jax.ShapeDtypeStruct((B, Z, YX), jnp.int32),
            jax.ShapeDtypeStruct((B, 1, 1), jnp.int32),
        ],
        scratch_shapes=[pltpu.VMEM((Z, YX), jnp.int32)],
        compiler_params=pltpu.CompilerParams(dimension_semantics=("parallel",)),
    )(tgt)

    d1t = d1.reshape(B, Z, Y, X).transpose(0, 2, 3, 1).reshape(B, Y, X * Z)

    d2 = pl.pallas_call(
        _minplus_kernel,
        grid=(B,),
        in_specs=[pl.BlockSpec((1, Y, X * Z), lambda b: (b, 0, 0))],
        out_specs=pl.BlockSpec((1, Y, X * Z), lambda b: (b, 0, 0)),
        out_shape=jax.ShapeDtypeStruct((B, Y, X * Z), jnp.int32),
        compiler_params=pltpu.CompilerParams(dimension_semantics=("parallel",)),
    )(d1t)

    d2t = d2.reshape(B, Y, X, Z).transpose(0, 2, 3, 1).reshape(B, X, Z * Y)
    pred_t = (
        pred.reshape(B, Z, Y, X).transpose(0, 3, 1, 2).reshape(B, X, Z * Y)
    )
    tgt_t = (
        target.reshape(B, Z, Y, X).transpose(0, 3, 1, 2).reshape(B, X, Z * Y)
    )

    losses = pl.pallas_call(
        _xpass_dice_kernel,
        grid=(B,),
        in_specs=[
            pl.BlockSpec((1, X, Z * Y), lambda b: (b, 0, 0)),
            pl.BlockSpec((1, X, Z * Y), lambda b: (b, 0, 0)),
            pl.BlockSpec((1, X, Z * Y), lambda b: (b, 0, 0)),
            pl.BlockSpec((1, 1, 1), lambda b: (b, 0, 0)),
        ],
        out_specs=pl.BlockSpec((1, 1, 1), lambda b: (b, 0, 0)),
        out_shape=jax.ShapeDtypeStruct((B, 1, 1), jnp.float32),
        scratch_shapes=[pltpu.VMEM((X, Z * Y), jnp.int32)],
        compiler_params=pltpu.CompilerParams(dimension_semantics=("parallel",)),
    )(d2t, pred_t, tgt_t, num)

    return jnp.mean(losses)


# speedup vs baseline: 9.7831x; 1.0615x over previous
import jax
import jax.numpy as jnp
import numpy as np
from jax import lax
from jax.experimental import pallas as pl
from jax.experimental.pallas import tpu as pltpu

B = 2
Z, Y, X = 64, 64, 64
YX = Y * X
R = B * 64
K = 1 << 18
INF = np.int32(2**31 - 1)
SIGN = np.int32(-(2**31))
BG_B = np.int32(3_200_000_000 - 2**31)
TILE = 512
CHECK_EVERY = 4


def _minplus_halves(src_ref, dst_ref):
    rvec = lax.broadcasted_iota(jnp.int32, (64, 1), 0)
    for h in range(0, R, 64):
        for t in range(0, YX, TILE):
            sl = pl.ds(t, TILE)
            m = None
            for j in range(64):
                row = src_ref[pl.ds(h + j, 1), sl]
                cost = (rvec - j) * (rvec - j) * K
                cand = row + cost
                m = cand if m is None else jnp.minimum(m, cand)
            dst_ref[pl.ds(h, 64), sl] = m


def _cc_and_zpass_kernel(target_ref, d1_ref, num_ref, lab_scr):
    tgt = target_ref[...]
    fg = tgt > 0.5
    colid = lax.broadcasted_iota(jnp.int32, (R, YX), 1)
    rowid = lax.broadcasted_iota(jnp.int32, (R, YX), 0)
    xpos = colid & (X - 1)
    ypos = colid >> 6
    zpos = rowid & 63
    flat = zpos * YX + colid

    lab_scr[...] = jnp.where(fg, flat, INF)

    inf_c1 = jnp.full((R, 1), INF, jnp.int32)
    inf_c64 = jnp.full((R, X), INF, jnp.int32)
    inf_r1 = jnp.full((1, YX), INF, jnp.int32)

    def prop(lab):
        a = jnp.concatenate([inf_c1, lab[:, :-1]], axis=1)
        a = jnp.where(xpos == 0, INF, a)
        b = jnp.concatenate([lab[:, 1:], inf_c1], axis=1)
        b = jnp.where(xpos == X - 1, INF, b)
        m = jnp.minimum(lab, jnp.minimum(a, b))
        a = jnp.concatenate([inf_c64, m[:, :-X]], axis=1)
        a = jnp.where(ypos == 0, INF, a)
        b = jnp.concatenate([m[:, X:], inf_c64], axis=1)
        b = jnp.where(ypos == Y - 1, INF, b)
        m = jnp.minimum(m, jnp.minimum(a, b))
        a = jnp.concatenate([inf_r1, m[:-1, :]], axis=0)
        a = jnp.where(zpos == 0, INF, a)
        b = jnp.concatenate([m[1:, :], inf_r1], axis=0)
        b = jnp.where(zpos == 63, INF, b)
        m = jnp.minimum(m, jnp.minimum(a, b))
        return jnp.where(fg, m, INF)

    def body(carry):
        cur = lab_scr[...]
        prev = cur
        for _ in range(CHECK_EVERY):
            prev = cur
            cur = prop(prev)
        lab_scr[...] = cur
        return jnp.max(jnp.where(cur != prev, 1, 0))

    lax.while_loop(lambda c: c > 0, lambda c: body(c), jnp.int32(1))

    lab = lab_scr[...]
    rep = (fg & (lab == flat)).astype(jnp.int32)
    num_ref[0] = jnp.sum(rep[:64]).reshape(1, 1)
    num_ref[1] = jnp.sum(rep[64:]).reshape(1, 1)

    lab_scr[...] = jnp.where(fg, lab ^ SIGN, BG_B)

    _minplus_halves(lab_scr, d1_ref)


def _minplus_kernel(din_ref, dout_ref):
    _minplus_halves(din_ref, dout_ref)


def _xpass_dice_kernel(din_ref, pred_ref, target_ref, num_ref, loss_ref, d_scr):
    _minplus_halves(din_ref, d_scr)

    ps = jax.nn.sigmoid(pred_ref[...])
    t = target_ref[...]
    pt = ps * t

    for b in range(B):
        root = d_scr[pl.ds(b * 64, 64), :] & (K - 1)
        psb = ps[b * 64 : (b + 1) * 64]
        tb = t[b * 64 : (b + 1) * 64]
        ptb = pt[b * 64 : (b + 1) * 64]
        num = num_ref[b, 0, 0]

        def dice_body(_, carry, root=root, psb=psb, tb=tb, ptb=ptb):
            proc, acc = carry
            r = jnp.min(jnp.where(root > proc, root, INF))
            maskf = (root == r).astype(jnp.float32)
            inter = jnp.sum(ptb * maskf)
            sp = jnp.sum(psb * maskf)
            sg = jnp.sum(tb * maskf)
            dice = 2.0 * inter / (sp + sg + 1e-8)
            return r, acc + dice

        _, ssum = lax.fori_loop(
            0, num, dice_body, (jnp.int32(-1), jnp.float32(0.0))
        )
        loss_ref[b] = jnp.where(
            num == 0, jnp.float32(1.0), 1.0 - ssum / num.astype(jnp.float32)
        ).reshape(1, 1)


def kernel(pred, target):
    tgt = target.reshape(R, YX)

    d1, num = pl.pallas_call(
        _cc_and_zpass_kernel,
        out_shape=[
            jax.ShapeDtypeStruct((R, YX), jnp.int32),
            jax.ShapeDtypeStruct((B, 1, 1), jnp.int32),
        ],
        scratch_shapes=[pltpu.VMEM((R, YX), jnp.int32)],
    )(tgt)

    d1t = d1.reshape(B, Z, Y, X).transpose(0, 2, 3, 1).reshape(R, X * Z)

    d2 = pl.pallas_call(
        _minplus_kernel,
        out_shape=jax.ShapeDtypeStruct((R, X * Z), jnp.int32),
    )(d1t)

    d2t = d2.reshape(B, Y, X, Z).transpose(0, 2, 3, 1).reshape(R, Z * Y)
    pred_t = pred.reshape(B, Z, Y, X).transpose(0, 3, 1, 2).reshape(R, Z * Y)
    tgt_t = target.reshape(B, Z, Y, X).transpose(0, 3, 1, 2).reshape(R, Z * Y)

    losses = pl.pallas_call(
        _xpass_dice_kernel,
        out_shape=jax.ShapeDtypeStruct((B, 1, 1), jnp.float32),
        scratch_shapes=[pltpu.VMEM((R, Z * Y), jnp.int32)],
    )(d2t, pred_t, tgt_t, num)

    return jnp.mean(losses)


# speedup vs baseline: 9.8765x; 1.0096x over previous
import jax
import jax.numpy as jnp
import numpy as np
from jax import lax
from jax.experimental import pallas as pl
from jax.experimental.pallas import tpu as pltpu

B = 2
Z, Y, X = 64, 64, 64
YX = Y * X
R = B * 64
K = 1 << 18
INF = np.int32(2**31 - 1)
SIGN = np.int32(-(2**31))
BG_B = np.int32(3_200_000_000 - 2**31)
TILE = 512
CHECK_EVERY = 4


def _minplus_halves(src_ref, dst_ref):
    rvec = lax.broadcasted_iota(jnp.int32, (64, 1), 0)
    for h in range(0, R, 64):
        for t in range(0, YX, TILE):
            sl = pl.ds(t, TILE)
            m = None
            for j in range(64):
                row = src_ref[pl.ds(h + j, 1), sl]
                cost = (rvec - j) * (rvec - j) * K
                cand = row + cost
                m = cand if m is None else jnp.minimum(m, cand)
            dst_ref[pl.ds(h, 64), sl] = m


def _cc_and_zpass_kernel(target_ref, d1_ref, num_ref, lab_scr):
    tgt = target_ref[...]
    fg = tgt > 0.5
    colid = lax.broadcasted_iota(jnp.int32, (R, YX), 1)
    rowid = lax.broadcasted_iota(jnp.int32, (R, YX), 0)
    xpos = colid & (X - 1)
    ypos = colid >> 6
    zpos = rowid & 63
    flat = zpos * YX + colid

    lab_scr[...] = jnp.where(fg, flat, INF)

    colid2 = lax.broadcasted_iota(jnp.int32, (2, YX), 1)
    xpos2 = colid2 & (X - 1)
    ypos2 = colid2 >> 6
    inf2_c1 = jnp.full((2, 1), INF, jnp.int32)
    inf2_c64 = jnp.full((2, X), INF, jnp.int32)
    inf_plane = jnp.full((2, YX), INF, jnp.int32)

    def min3x3(v):
        a = jnp.concatenate([inf2_c1, v[:, :-1]], axis=1)
        a = jnp.where(xpos2 == 0, INF, a)
        b = jnp.concatenate([v[:, 1:], inf2_c1], axis=1)
        b = jnp.where(xpos2 == X - 1, INF, b)
        m = jnp.minimum(v, jnp.minimum(a, b))
        a = jnp.concatenate([inf2_c64, m[:, :-X]], axis=1)
        a = jnp.where(ypos2 == 0, INF, a)
        b = jnp.concatenate([m[:, X:], inf2_c64], axis=1)
        b = jnp.where(ypos2 == Y - 1, INF, b)
        return jnp.minimum(m, jnp.minimum(a, b))

    def sweep_step(z, carry):
        prev, changed = carry
        cur = jnp.concatenate(
            [lab_scr[pl.ds(z, 1), :], lab_scr[pl.ds(z + 64, 1), :]], axis=0
        )
        fgrow = jnp.concatenate(
            [target_ref[pl.ds(z, 1), :], target_ref[pl.ds(z + 64, 1), :]],
            axis=0,
        ) > 0.5
        new = jnp.where(
            fgrow,
            jnp.minimum(cur, jnp.minimum(min3x3(prev), min3x3(cur))),
            INF,
        )
        lab_scr[pl.ds(z, 1), :] = new[0:1]
        lab_scr[pl.ds(z + 64, 1), :] = new[1:2]
        changed = jnp.maximum(
            changed, jnp.max(jnp.where(new != cur, 1, 0))
        )
        return new, changed

    def body(carry):
        _, changed = lax.fori_loop(
            0, 64, lambda s, c: sweep_step(s, c), (inf_plane, jnp.int32(0))
        )
        _, changed = lax.fori_loop(
            0, 64, lambda s, c: sweep_step(63 - s, c), (inf_plane, changed)
        )
        return changed

    lax.while_loop(lambda c: c > 0, lambda c: body(c), jnp.int32(1))

    lab = lab_scr[...]
    rep = (fg & (lab == flat)).astype(jnp.int32)
    num_ref[0] = jnp.sum(rep[:64]).reshape(1, 1)
    num_ref[1] = jnp.sum(rep[64:]).reshape(1, 1)

    lab_scr[...] = jnp.where(fg, lab ^ SIGN, BG_B)

    _minplus_halves(lab_scr, d1_ref)


def _minplus_kernel(din_ref, dout_ref):
    _minplus_halves(din_ref, dout_ref)


def _xpass_dice_kernel(din_ref, pred_ref, target_ref, num_ref, loss_ref, d_scr):
    _minplus_halves(din_ref, d_scr)

    ps = jax.nn.sigmoid(pred_ref[...])
    t = target_ref[...]
    pt = ps * t

    for b in range(B):
        root = d_scr[pl.ds(b * 64, 64), :] & (K - 1)
        psb = ps[b * 64 : (b + 1) * 64]
        tb = t[b * 64 : (b + 1) * 64]
        ptb = pt[b * 64 : (b + 1) * 64]
        num = num_ref[b, 0, 0]

        def dice_body(_, carry, root=root, psb=psb, tb=tb, ptb=ptb):
            proc, acc = carry
            r = jnp.min(jnp.where(root > proc, root, INF))
            maskf = (root == r).astype(jnp.float32)
            inter = jnp.sum(ptb * maskf)
            sp = jnp.sum(psb * maskf)
            sg = jnp.sum(tb * maskf)
            dice = 2.0 * inter / (sp + sg + 1e-8)
            return r, acc + dice

        _, ssum = lax.fori_loop(
            0, num, dice_body, (jnp.int32(-1), jnp.float32(0.0))
        )
        loss_ref[b] = jnp.where(
            num == 0, jnp.float32(1.0), 1.0 - ssum / num.astype(jnp.float32)
        ).reshape(1, 1)


def kernel(pred, target):
    tgt = target.reshape(R, YX)

    d1, num = pl.pallas_call(
        _cc_and_zpass_kernel,
        out_shape=[
            jax.ShapeDtypeStruct((R, YX), jnp.int32),
            jax.ShapeDtypeStruct((B, 1, 1), jnp.int32),
        ],
        scratch_shapes=[pltpu.VMEM((R, YX), jnp.int32)],
    )(tgt)

    d1t = d1.reshape(B, Z, Y, X).transpose(0, 2, 3, 1).reshape(R, X * Z)

    d2 = pl.pallas_call(
        _minplus_kernel,
        out_shape=jax.ShapeDtypeStruct((R, X * Z), jnp.int32),
    )(d1t)

    d2t = d2.reshape(B, Y, X, Z).transpose(0, 2, 3, 1).reshape(R, Z * Y)
    pred_t = pred.reshape(B, Z, Y, X).transpose(0, 3, 1, 2).reshape(R, Z * Y)
    tgt_t = target.reshape(B, Z, Y, X).transpose(0, 3, 1, 2).reshape(R, Z * Y)

    losses = pl.pallas_call(
        _xpass_dice_kernel,
        out_shape=jax.ShapeDtypeStruct((B, 1, 1), jnp.float32),
        scratch_shapes=[pltpu.VMEM((R, Z * Y), jnp.int32)],
    )(d2t, pred_t, tgt_t, num)

    return jnp.mean(losses)


# speedup vs baseline: 14.9549x; 1.5142x over previous
import jax
import jax.numpy as jnp
import numpy as np
from jax import lax
from jax.experimental import pallas as pl
from jax.experimental.pallas import tpu as pltpu

B = 2
Z, Y, X = 64, 64, 64
YX = Y * X
R = B * 64
K = 1 << 18
INF = np.int32(2**31 - 1)
SIGN = np.int32(-(2**31))
BG_B = np.int32(3_200_000_000 - 2**31)
TILE = 512
CHECK_EVERY = 4


def _minplus_halves(src_ref, dst_ref):
    rvec = lax.broadcasted_iota(jnp.int32, (64, 1), 0)
    for h in range(0, R, 64):
        for t in range(0, YX, TILE):
            sl = pl.ds(t, TILE)
            m = None
            for j in range(64):
                row = src_ref[pl.ds(h + j, 1), sl]
                cost = (rvec - j) * (rvec - j) * K
                cand = row + cost
                m = cand if m is None else jnp.minimum(m, cand)
            dst_ref[pl.ds(h, 64), sl] = m


def _cc_and_zpass_kernel(target_ref, d1_ref, num_ref, lab_scr, bg_scr):
    tgt = target_ref[...]
    fg = tgt > 0.5
    colid = lax.broadcasted_iota(jnp.int32, (R, YX), 1)
    rowid = lax.broadcasted_iota(jnp.int32, (R, YX), 0)
    zpos = rowid & 63
    flat = zpos * YX + colid

    lab_scr[...] = jnp.where(fg, flat, INF)
    bg_scr[...] = jnp.where(fg, 0, INF)

    colid4 = lax.broadcasted_iota(jnp.int32, (4, YX), 1)
    xpos4 = colid4 & (X - 1)
    ypos4 = colid4 >> 6
    inf4_c1 = jnp.full((4, 1), INF, jnp.int32)
    inf4_c64 = jnp.full((4, X), INF, jnp.int32)
    inf_slab = jnp.full((4, YX), INF, jnp.int32)

    def min3x3(v):
        a = jnp.concatenate([inf4_c1, v[:, :-1]], axis=1)
        a = jnp.where(xpos4 == 0, INF, a)
        b = jnp.concatenate([v[:, 1:], inf4_c1], axis=1)
        b = jnp.where(xpos4 == X - 1, INF, b)
        m = jnp.minimum(v, jnp.minimum(a, b))
        a = jnp.concatenate([inf4_c64, m[:, :-X]], axis=1)
        a = jnp.where(ypos4 == 0, INF, a)
        b = jnp.concatenate([m[:, X:], inf4_c64], axis=1)
        b = jnp.where(ypos4 == Y - 1, INF, b)
        return jnp.minimum(m, jnp.minimum(a, b))

    def body(carry):
        old = lab_scr[...]
        prev = inf_slab
        for s in range(64):
            rows = (s, s + 64, 63 - s, 127 - s)
            cur = jnp.concatenate(
                [lab_scr[pl.ds(r, 1), :] for r in rows], axis=0
            )
            floor = jnp.concatenate(
                [bg_scr[pl.ds(r, 1), :] for r in rows], axis=0
            )
            new = jnp.maximum(
                jnp.minimum(cur, jnp.minimum(min3x3(prev), min3x3(cur))),
                floor,
            )
            for i, r in enumerate(rows):
                lab_scr[pl.ds(r, 1), :] = new[i : i + 1]
            prev = new
        return jnp.max(jnp.where(lab_scr[...] != old, 1, 0))

    lax.while_loop(lambda c: c > 0, lambda c: body(c), jnp.int32(1))

    lab = lab_scr[...]
    rep = (fg & (lab == flat)).astype(jnp.int32)
    num_ref[0] = jnp.sum(rep[:64]).reshape(1, 1)
    num_ref[1] = jnp.sum(rep[64:]).reshape(1, 1)

    lab_scr[...] = jnp.where(fg, lab ^ SIGN, BG_B)

    _minplus_halves(lab_scr, d1_ref)


def _minplus_kernel(din_ref, dout_ref):
    _minplus_halves(din_ref, dout_ref)


def _xpass_dice_kernel(din_ref, pred_ref, target_ref, num_ref, loss_ref, d_scr):
    _minplus_halves(din_ref, d_scr)

    ps = jax.nn.sigmoid(pred_ref[...])
    t = target_ref[...]
    pt = ps * t

    for b in range(B):
        root = d_scr[pl.ds(b * 64, 64), :] & (K - 1)
        psb = ps[b * 64 : (b + 1) * 64]
        tb = t[b * 64 : (b + 1) * 64]
        ptb = pt[b * 64 : (b + 1) * 64]
        num = num_ref[b, 0, 0]

        def dice_body(_, carry, root=root, psb=psb, tb=tb, ptb=ptb):
            proc, acc = carry
            r = jnp.min(jnp.where(root > proc, root, INF))
            maskf = (root == r).astype(jnp.float32)
            inter = jnp.sum(ptb * maskf)
            sp = jnp.sum(psb * maskf)
            sg = jnp.sum(tb * maskf)
            dice = 2.0 * inter / (sp + sg + 1e-8)
            return r, acc + dice

        _, ssum = lax.fori_loop(
            0, num, dice_body, (jnp.int32(-1), jnp.float32(0.0))
        )
        loss_ref[b] = jnp.where(
            num == 0, jnp.float32(1.0), 1.0 - ssum / num.astype(jnp.float32)
        ).reshape(1, 1)


def kernel(pred, target):
    tgt = target.reshape(R, YX)

    d1, num = pl.pallas_call(
        _cc_and_zpass_kernel,
        out_shape=[
            jax.ShapeDtypeStruct((R, YX), jnp.int32),
            jax.ShapeDtypeStruct((B, 1, 1), jnp.int32),
        ],
        scratch_shapes=[pltpu.VMEM((R, YX), jnp.int32),
                        pltpu.VMEM((R, YX), jnp.int32)],
    )(tgt)

    d1t = d1.reshape(B, Z, Y, X).transpose(0, 2, 3, 1).reshape(R, X * Z)

    d2 = pl.pallas_call(
        _minplus_kernel,
        out_shape=jax.ShapeDtypeStruct((R, X * Z), jnp.int32),
    )(d1t)

    d2t = d2.reshape(B, Y, X, Z).transpose(0, 2, 3, 1).reshape(R, Z * Y)
    pred_t = pred.reshape(B, Z, Y, X).transpose(0, 3, 1, 2).reshape(R, Z * Y)
    tgt_t = target.reshape(B, Z, Y, X).transpose(0, 3, 1, 2).reshape(R, Z * Y)

    losses = pl.pallas_call(
        _xpass_dice_kernel,
        out_shape=jax.ShapeDtypeStruct((B, 1, 1), jnp.float32),
        scratch_shapes=[pltpu.VMEM((R, Z * Y), jnp.int32)],
    )(d2t, pred_t, tgt_t, num)

    return jnp.mean(losses)


# speedup vs baseline: 14.9669x; 1.0008x over previous
import jax
import jax.numpy as jnp
import numpy as np
from jax import lax
from jax.experimental import pallas as pl
from jax.experimental.pallas import tpu as pltpu

B = 2
Z, Y, X = 64, 64, 64
YX = Y * X
R = B * 64
K = 1 << 18
INF = np.int32(2**31 - 1)
SIGN = np.int32(-(2**31))
BG_B = np.int32(3_200_000_000 - 2**31)
TILE = 512
CHECK_EVERY = 4


def _minplus_halves(src_ref, dst_ref):
    rvec = lax.broadcasted_iota(jnp.int32, (64, 1), 0)
    for h in range(0, R, 64):
        for t in range(0, YX, TILE):
            sl = pl.ds(t, TILE)
            m = None
            for j in range(64):
                row = src_ref[pl.ds(h + j, 1), sl]
                cost = (rvec - j) * (rvec - j) * K
                cand = row + cost
                m = cand if m is None else jnp.minimum(m, cand)
            dst_ref[pl.ds(h, 64), sl] = m


def _cc_and_zpass_kernel(target_ref, d1_ref, num_ref, lab_scr):
    tgt = target_ref[...]
    fg = tgt > 0.5
    colid = lax.broadcasted_iota(jnp.int32, (R, YX), 1)
    rowid = lax.broadcasted_iota(jnp.int32, (R, YX), 0)
    parity = rowid & 1
    flat = (rowid >> 1) * YX + colid

    lab_scr[...] = jnp.where(fg, flat, INF)

    colid4 = lax.broadcasted_iota(jnp.int32, (4, YX), 1)
    xpos4 = colid4 & (X - 1)
    ypos4 = colid4 >> 6
    inf4_c1 = jnp.full((4, 1), INF, jnp.int32)
    inf4_c64 = jnp.full((4, X), INF, jnp.int32)
    inf_slab = jnp.full((4, YX), INF, jnp.int32)

    def min3x3(v):
        a = jnp.concatenate([inf4_c1, v[:, :-1]], axis=1)
        a = jnp.where(xpos4 == 0, INF, a)
        b = jnp.concatenate([v[:, 1:], inf4_c1], axis=1)
        b = jnp.where(xpos4 == X - 1, INF, b)
        m = jnp.minimum(v, jnp.minimum(a, b))
        a = jnp.concatenate([inf4_c64, m[:, :-X]], axis=1)
        a = jnp.where(ypos4 == 0, INF, a)
        b = jnp.concatenate([m[:, X:], inf4_c64], axis=1)
        b = jnp.where(ypos4 == Y - 1, INF, b)
        return jnp.minimum(m, jnp.minimum(a, b))

    def body(carry):
        old = lab_scr[...]
        prev = inf_slab
        for s in range(64):
            fsl = pl.ds(2 * s, 2)
            bsl = pl.ds(2 * (63 - s), 2)
            cur = jnp.concatenate([lab_scr[fsl, :], lab_scr[bsl, :]], axis=0)
            cand = jnp.minimum(cur, jnp.minimum(min3x3(prev), min3x3(cur)))
            new = jnp.where(cur != INF, cand, INF)
            lab_scr[fsl, :] = new[0:2]
            lab_scr[bsl, :] = new[2:4]
            prev = new
        return jnp.max(jnp.where(lab_scr[...] != old, 1, 0))

    lax.while_loop(lambda c: c > 0, lambda c: body(c), jnp.int32(1))

    lab = lab_scr[...]
    rep = (fg & (lab == flat)).astype(jnp.int32)
    num_ref[0] = jnp.sum(rep * (1 - parity)).reshape(1, 1)
    num_ref[1] = jnp.sum(rep * parity).reshape(1, 1)

    lab_scr[...] = jnp.where(fg, lab ^ SIGN, BG_B)

    rvec = lax.broadcasted_iota(jnp.int32, (64, 1), 0)
    for b in range(B):
        for t in range(0, YX, TILE):
            sl = pl.ds(t, TILE)
            m = None
            for j in range(64):
                row = lab_scr[pl.ds(2 * j + b, 1), sl]
                cost = (rvec - j) * (rvec - j) * K
                cand = row + cost
                m = cand if m is None else jnp.minimum(m, cand)
            d1_ref[pl.ds(b * 64, 64), sl] = m


def _minplus_kernel(din_ref, dout_ref):
    _minplus_halves(din_ref, dout_ref)


def _xpass_dice_kernel(din_ref, pred_ref, target_ref, num_ref, loss_ref, d_scr):
    _minplus_halves(din_ref, d_scr)

    ps = jax.nn.sigmoid(pred_ref[...])
    t = target_ref[...]
    pt = ps * t

    for b in range(B):
        root = d_scr[pl.ds(b * 64, 64), :] & (K - 1)
        psb = ps[b * 64 : (b + 1) * 64]
        tb = t[b * 64 : (b + 1) * 64]
        ptb = pt[b * 64 : (b + 1) * 64]
        num = num_ref[b, 0, 0]

        def dice_body(_, carry, root=root, psb=psb, tb=tb, ptb=ptb):
            proc, acc = carry
            r = jnp.min(jnp.where(root > proc, root, INF))
            maskf = (root == r).astype(jnp.float32)
            inter = jnp.sum(ptb * maskf)
            sp = jnp.sum(psb * maskf)
            sg = jnp.sum(tb * maskf)
            dice = 2.0 * inter / (sp + sg + 1e-8)
            return r, acc + dice

        _, ssum = lax.fori_loop(
            0, num, dice_body, (jnp.int32(-1), jnp.float32(0.0))
        )
        loss_ref[b] = jnp.where(
            num == 0, jnp.float32(1.0), 1.0 - ssum / num.astype(jnp.float32)
        ).reshape(1, 1)


def kernel(pred, target):
    tgt = target.reshape(B, Z, YX).transpose(1, 0, 2).reshape(R, YX)

    d1, num = pl.pallas_call(
        _cc_and_zpass_kernel,
        out_shape=[
            jax.ShapeDtypeStruct((R, YX), jnp.int32),
            jax.ShapeDtypeStruct((B, 1, 1), jnp.int32),
        ],
        scratch_shapes=[pltpu.VMEM((R, YX), jnp.int32)],
    )(tgt)

    d1t = d1.reshape(B, Z, Y, X).transpose(0, 2, 3, 1).reshape(R, X * Z)

    d2 = pl.pallas_call(
        _minplus_kernel,
        out_shape=jax.ShapeDtypeStruct((R, X * Z), jnp.int32),
    )(d1t)

    d2t = d2.reshape(B, Y, X, Z).transpose(0, 2, 3, 1).reshape(R, Z * Y)
    pred_t = pred.reshape(B, Z, Y, X).transpose(0, 3, 1, 2).reshape(R, Z * Y)
    tgt_t = target.reshape(B, Z, Y, X).transpose(0, 3, 1, 2).reshape(R, Z * Y)

    losses = pl.pallas_call(
        _xpass_dice_kernel,
        out_shape=jax.ShapeDtypeStruct((B, 1, 1), jnp.float32),
        scratch_shapes=[pltpu.VMEM((R, Z * Y), jnp.int32)],
    )(d2t, pred_t, tgt_t, num)

    return jnp.mean(losses)


# speedup vs baseline: 19.9284x; 1.3315x over previous
import jax
import jax.numpy as jnp
import numpy as np
from jax import lax
from jax.experimental import pallas as pl
from jax.experimental.pallas import tpu as pltpu

B = 2
Z, Y, X = 64, 64, 64
YX = Y * X
R = B * 64
K = 1 << 18
INF = np.int32(2**31 - 1)
SIGN = np.int32(-(2**31))
BG_B = np.int32(3_200_000_000 - 2**31)
TILE = 512
CHECK_EVERY = 4


def _minplus_halves(src_ref, dst_ref):
    rvec = lax.broadcasted_iota(jnp.int32, (64, 1), 0)
    for h in range(0, R, 64):
        for t in range(0, YX, TILE):
            sl = pl.ds(t, TILE)
            m = None
            for j in range(64):
                row = src_ref[pl.ds(h + j, 1), sl]
                cost = (rvec - j) * (rvec - j) * K
                cand = row + cost
                m = cand if m is None else jnp.minimum(m, cand)
            dst_ref[pl.ds(h, 64), sl] = m


def _cc_and_zpass_kernel(target_ref, d1_ref, num_ref, lab_scr):
    tgt = target_ref[...]
    fg = tgt > 0.5
    colid = lax.broadcasted_iota(jnp.int32, (R, YX), 1)
    rowid = lax.broadcasted_iota(jnp.int32, (R, YX), 0)
    parity = rowid & 1
    flat = (rowid >> 1) * YX + colid

    lab_scr[...] = jnp.where(fg, flat, INF)

    colid4 = lax.broadcasted_iota(jnp.int32, (4, YX), 1)
    xpos4 = colid4 & (X - 1)
    ypos4 = colid4 >> 6
    inf4_c1 = jnp.full((4, 1), INF, jnp.int32)
    inf4_c64 = jnp.full((4, X), INF, jnp.int32)
    inf_slab = jnp.full((4, YX), INF, jnp.int32)

    def min3x3(v):
        a = jnp.concatenate([inf4_c1, v[:, :-1]], axis=1)
        a = jnp.where(xpos4 == 0, INF, a)
        b = jnp.concatenate([v[:, 1:], inf4_c1], axis=1)
        b = jnp.where(xpos4 == X - 1, INF, b)
        m = jnp.minimum(v, jnp.minimum(a, b))
        a = jnp.concatenate([inf4_c64, m[:, :-X]], axis=1)
        a = jnp.where(ypos4 == 0, INF, a)
        b = jnp.concatenate([m[:, X:], inf4_c64], axis=1)
        b = jnp.where(ypos4 == Y - 1, INF, b)
        return jnp.minimum(m, jnp.minimum(a, b))

    def body(carry):
        old = lab_scr[...]
        prev = inf_slab
        for s in range(64):
            fsl = pl.ds(2 * s, 2)
            bsl = pl.ds(2 * (63 - s), 2)
            cur = jnp.concatenate([lab_scr[fsl, :], lab_scr[bsl, :]], axis=0)
            cand = min3x3(jnp.minimum(prev, cur))
            new = jnp.where(cur != INF, cand, INF)
            lab_scr[fsl, :] = new[0:2]
            lab_scr[bsl, :] = new[2:4]
            prev = new
        return jnp.max(jnp.where(lab_scr[...] != old, 1, 0))

    lax.while_loop(lambda c: c > 0, lambda c: body(c), jnp.int32(1))

    lab = lab_scr[...]
    rep = (fg & (lab == flat)).astype(jnp.int32)
    num_ref[0] = jnp.sum(rep * (1 - parity)).reshape(1, 1)
    num_ref[1] = jnp.sum(rep * parity).reshape(1, 1)

    lab_scr[...] = jnp.where(fg, lab ^ SIGN, BG_B)

    rvec = lax.broadcasted_iota(jnp.int32, (64, 1), 0)
    for b in range(B):
        for t in range(0, YX, TILE):
            sl = pl.ds(t, TILE)
            m = None
            for j in range(64):
                row = lab_scr[pl.ds(2 * j + b, 1), sl]
                cost = (rvec - j) * (rvec - j) * K
                cand = row + cost
                m = cand if m is None else jnp.minimum(m, cand)
            d1_ref[pl.ds(b * 64, 64), sl] = m


def _minplus_kernel(din_ref, dout_ref):
    _minplus_halves(din_ref, dout_ref)


def _xpass_dice_kernel(din_ref, pred_ref, target_ref, num_ref, loss_ref, d_scr):
    _minplus_halves(din_ref, d_scr)

    ps = jax.nn.sigmoid(pred_ref[...])
    t = target_ref[...]
    pt = ps * t

    for b in range(B):
        root = d_scr[pl.ds(b * 64, 64), :] & (K - 1)
        psb = ps[b * 64 : (b + 1) * 64]
        tb = t[b * 64 : (b + 1) * 64]
        ptb = pt[b * 64 : (b + 1) * 64]
        num = num_ref[b, 0, 0]

        def dice_body(_, carry, root=root, psb=psb, tb=tb, ptb=ptb):
            proc, acc = carry
            r = jnp.min(jnp.where(root > proc, root, INF))
            maskf = (root == r).astype(jnp.float32)
            inter = jnp.sum(ptb * maskf)
            sp = jnp.sum(psb * maskf)
            sg = jnp.sum(tb * maskf)
            dice = 2.0 * inter / (sp + sg + 1e-8)
            return r, acc + dice

        _, ssum = lax.fori_loop(
            0, num, dice_body, (jnp.int32(-1), jnp.float32(0.0))
        )
        loss_ref[b] = jnp.where(
            num == 0, jnp.float32(1.0), 1.0 - ssum / num.astype(jnp.float32)
        ).reshape(1, 1)


def kernel(pred, target):
    tgt = target.reshape(B, Z, YX).transpose(1, 0, 2).reshape(R, YX)

    d1, num = pl.pallas_call(
        _cc_and_zpass_kernel,
        out_shape=[
            jax.ShapeDtypeStruct((R, YX), jnp.int32),
            jax.ShapeDtypeStruct((B, 1, 1), jnp.int32),
        ],
        scratch_shapes=[pltpu.VMEM((R, YX), jnp.int32)],
    )(tgt)

    d1t = d1.reshape(B, Z, Y, X).transpose(0, 2, 3, 1).reshape(R, X * Z)

    d2 = pl.pallas_call(
        _minplus_kernel,
        out_shape=jax.ShapeDtypeStruct((R, X * Z), jnp.int32),
    )(d1t)

    d2t = d2.reshape(B, Y, X, Z).transpose(0, 2, 3, 1).reshape(R, Z * Y)
    pred_t = pred.reshape(B, Z, Y, X).transpose(0, 3, 1, 2).reshape(R, Z * Y)
    tgt_t = target.reshape(B, Z, Y, X).transpose(0, 3, 1, 2).reshape(R, Z * Y)

    losses = pl.pallas_call(
        _xpass_dice_kernel,
        out_shape=jax.ShapeDtypeStruct((B, 1, 1), jnp.float32),
        scratch_shapes=[pltpu.VMEM((R, Z * Y), jnp.int32)],
    )(d2t, pred_t, tgt_t, num)

    return jnp.mean(losses)
